# baseline (speedup 1.0000x reference)
import jax
import jax.numpy as jnp
from jax import lax
from jax.experimental import pallas as pl
from jax.experimental.pallas import tpu as pltpu
from jax.experimental.pallas import tpu_sc as plsc

B = 1024
J = 2048
N = 4096
K = 12
NC = 2
NS = 16
LANES = 16

N_PER_SC = N // NC
B_PER_TILE = B // NS
N_PER_TILE = N_PER_SC // NS
CONN_BLK = 256
N_CHUNK = 32
GROUPS = B // (4 * LANES)
WORDS = B // 4


def _sc_body(t3_hbm, conn_hbm, mem_hbm, out_hbm, addr_sh):
    c = lax.axis_index("c")
    s = lax.axis_index("s")
    lane = lax.iota(jnp.int32, LANES)

    def phase1(inp_v, conn_v, accbuf):
        pltpu.sync_copy(t3_hbm.at[pl.ds(pl.multiple_of(s * LANES, 8), LANES), :],
                        inp_v.at[:, pl.ds(0, J)])

        def blk_body(blk, _):
            n0 = c * N_PER_SC + blk * CONN_BLK
            pltpu.sync_copy(conn_hbm.at[pl.ds(pl.multiple_of(n0 // LANES, 8), CONN_BLK // LANES), :],
                            conn_v)

            @plsc.parallel_loop(0, CONN_BLK, unroll=4)
            def n_body(nn):
                hi = jnp.zeros((LANES,), jnp.int32)
                lo = jnp.zeros((LANES,), jnp.int32)
                cvec = conn_v[nn >> 4, pl.ds((nn & 15) * LANES, LANES)]
                for k in range(K):
                    ckb = jnp.zeros((LANES,), jnp.int32) + cvec[k]
                    wv = plsc.load_gather(inp_v, [lane, ckb])
                    if k < 6:
                        hi = hi + (wv << (5 - k))
                    else:
                        lo = lo + (wv << (11 - k))
                accbuf[nn, pl.ds(0, LANES)] = hi
                accbuf[nn, pl.ds(LANES, LANES)] = lo
            pltpu.sync_copy(
                accbuf,
                addr_sh.at[pl.ds((blk & 1) * CONN_BLK, CONN_BLK), blk >> 1, s, :])
            return 0

        lax.fori_loop(0, N_PER_SC // CONN_BLK, blk_body, 0)

    pl.run_scoped(
        phase1,
        pltpu.VMEM((LANES, J + 1), jnp.int32),
        pltpu.VMEM((CONN_BLK // LANES, LANES * LANES), jnp.int32),
        pltpu.VMEM((CONN_BLK, 2 * LANES), jnp.int32),
    )
    plsc.subcore_barrier()

    rbase = c * (N_PER_SC // 4) + s * N_CHUNK

    def phase2(mem_v, addr_v, out_v):
        def chunk_body(j, _):
            pltpu.sync_copy(
                mem_hbm.at[pl.ds(pl.multiple_of(rbase + j * 8, 8), 8), :],
                mem_v)
            pltpu.sync_copy(
                addr_sh.at[pl.ds(s * N_CHUNK + j * 8, 8), :, :, :], addr_v)

            def i_body(i, _):
                row = jnp.zeros((LANES,), jnp.int32) + (i >> 2)
                nsh = (i & 3) * 8

                @plsc.parallel_loop(0, GROUPS, unroll=4)
                def g_body(g):
                    hi = addr_v[i >> 2, i & 3, g, pl.ds(0, LANES)]
                    lo = addr_v[i >> 2, i & 3, g, pl.ds(LANES, LANES)]
                    out_w = jnp.zeros((LANES,), jnp.int32)
                    for bi in range(4):
                        h = (hi >> (8 * bi)) & 63
                        l = (lo >> (8 * bi)) & 63
                        a = (h << 6) | l
                        wv = plsc.load_gather(mem_v, [row, a])
                        byte = (wv >> nsh) & 255
                        r = (byte == 2).astype(jnp.int32)
                        out_w = out_w | (r << (8 * bi))
                    out_v[i & 3, i >> 2, pl.ds(g * LANES, LANES)] = out_w
                return 0

            lax.fori_loop(0, N_CHUNK, i_body, 0)
            pltpu.sync_copy(
                out_v,
                out_hbm.at[c, :, pl.ds(s * N_CHUNK + j * 8, 8), :])
            return 0

        lax.fori_loop(0, N_PER_TILE // N_CHUNK, chunk_body, 0)

    pl.run_scoped(
        phase2,
        pltpu.VMEM((8, 4096), jnp.int32),
        pltpu.VMEM((8, 4, GROUPS, 2 * LANES), jnp.int32),
        pltpu.VMEM((4, 8, WORDS), jnp.int32),
    )


def _sc_call(t3, conn, mem):
    mesh = plsc.VectorSubcoreMesh(core_axis_name="c", subcore_axis_name="s")
    return pl.kernel(
        _sc_body,
        out_type=jax.ShapeDtypeStruct((NC, 4, N_PER_SC // 4, WORDS), jnp.int32),
        name="ramlayer_sc",
        mesh=mesh,
        compiler_params=pltpu.CompilerParams(
            needs_layout_passes=False,
            use_tc_tiling_on_sc=False,
        ),
        scratch_types=[
            pltpu.VMEM_SHARED((N_PER_SC // 4, 4, NS, 2 * LANES), jnp.int32),
        ],
    )(t3, conn, mem)


def _pack4(p0, p1, p2, p3):
    return (p0.astype(jnp.int32) | (p1.astype(jnp.int32) << 8)
            | (p2.astype(jnp.int32) << 16) | (p3.astype(jnp.int32) << 24))


def kernel(input_bits, connections, memory):
    ib = input_bits.astype(jnp.uint8)
    t3w = _pack4(ib[0:256], ib[256:512], ib[512:768], ib[768:1024])
    mem32 = jnp.concatenate(
        [_pack4(*(memory[c * 2048 + 512 * q: c * 2048 + 512 * (q + 1)]
                  for q in range(4))) for c in range(NC)], axis=0)
    conn_p = jnp.pad(connections, ((0, 0), (0, LANES - K)))
    conn_f = conn_p.reshape(N // LANES, LANES * LANES)
    outw = _sc_call(t3w, conn_f, mem32)
    outw = outw.reshape(N, WORDS).T
    ys = [((outw >> (8 * i)) & 1).astype(jnp.bool_) for i in range(4)]
    return jnp.concatenate(ys, axis=0)


# speedup vs baseline: 1.0556x; 1.0556x over previous
import jax
import jax.numpy as jnp
from jax import lax
from jax.experimental import pallas as pl
from jax.experimental.pallas import tpu as pltpu
from jax.experimental.pallas import tpu_sc as plsc

B = 1024
J = 2048
N = 4096
K = 12
NC = 2
NS = 16
LANES = 16

N_PER_SC = N // NC
B_PER_TILE = B // NS
N_PER_TILE = N_PER_SC // NS
CONN_BLK = 256
N_CHUNK = 32
GROUPS = B // (4 * LANES)
WORDS = B // 4


def _sc_body(t3_hbm, conn_hbm, mem_hbm, out_hbm, addr_sh):
    c = lax.axis_index("c")
    s = lax.axis_index("s")
    lane = lax.iota(jnp.int32, LANES)

    def phase1(inp_v, conn_v, accbuf):
        pltpu.sync_copy(t3_hbm.at[:, pl.ds(s * LANES, LANES)], inp_v)

        def blk_body(blk, _):
            n0 = c * N_PER_SC + blk * CONN_BLK
            pltpu.sync_copy(conn_hbm.at[pl.ds(pl.multiple_of(n0 // LANES, 8), CONN_BLK // LANES), :],
                            conn_v)

            @plsc.parallel_loop(0, CONN_BLK, unroll=4)
            def n_body(nn):
                hi = jnp.zeros((LANES,), jnp.int32)
                lo = jnp.zeros((LANES,), jnp.int32)
                cvec = conn_v[nn >> 4, pl.ds((nn & 15) * LANES, LANES)]
                for k in range(K):
                    wv = inp_v[cvec[k], pl.ds(0, LANES)]
                    if k < 6:
                        hi = hi + (wv << (5 - k))
                    else:
                        lo = lo + (wv << (11 - k))
                accbuf[nn, pl.ds(0, LANES)] = hi
                accbuf[nn, pl.ds(LANES, LANES)] = lo
            pltpu.sync_copy(
                accbuf,
                addr_sh.at[pl.ds((blk & 1) * CONN_BLK, CONN_BLK), blk >> 1, s, :])
            return 0

        lax.fori_loop(0, N_PER_SC // CONN_BLK, blk_body, 0)

    pl.run_scoped(
        phase1,
        pltpu.VMEM((J, LANES), jnp.int32),
        pltpu.VMEM((CONN_BLK // LANES, LANES * LANES), jnp.int32),
        pltpu.VMEM((CONN_BLK, 2 * LANES), jnp.int32),
    )
    plsc.subcore_barrier()

    rbase = c * (N_PER_SC // 4) + s * N_CHUNK

    def phase2(mem_v, addr_v, out_v):
        def chunk_body(j, _):
            pltpu.sync_copy(
                mem_hbm.at[pl.ds(pl.multiple_of(rbase + j * 8, 8), 8), :],
                mem_v)
            pltpu.sync_copy(
                addr_sh.at[pl.ds(s * N_CHUNK + j * 8, 8), :, :, :], addr_v)

            def i_body(i, _):
                row = jnp.zeros((LANES,), jnp.int32) + (i >> 2)
                nsh = (i & 3) * 8

                @plsc.parallel_loop(0, GROUPS, unroll=4)
                def g_body(g):
                    hi = addr_v[i >> 2, i & 3, g, pl.ds(0, LANES)]
                    lo = addr_v[i >> 2, i & 3, g, pl.ds(LANES, LANES)]
                    out_w = jnp.zeros((LANES,), jnp.int32)
                    for bi in range(4):
                        h = (hi >> (8 * bi)) & 63
                        l = (lo >> (8 * bi)) & 63
                        a = (h << 6) | l
                        wv = plsc.load_gather(mem_v, [row, a])
                        byte = (wv >> nsh) & 255
                        r = (byte == 2).astype(jnp.int32)
                        out_w = out_w | (r << (8 * bi))
                    out_v[i & 3, i >> 2, pl.ds(g * LANES, LANES)] = out_w
                return 0

            lax.fori_loop(0, N_CHUNK, i_body, 0)
            pltpu.sync_copy(
                out_v,
                out_hbm.at[c, :, pl.ds(s * N_CHUNK + j * 8, 8), :])
            return 0

        lax.fori_loop(0, N_PER_TILE // N_CHUNK, chunk_body, 0)

    pl.run_scoped(
        phase2,
        pltpu.VMEM((8, 4096), jnp.int32),
        pltpu.VMEM((8, 4, GROUPS, 2 * LANES), jnp.int32),
        pltpu.VMEM((4, 8, WORDS), jnp.int32),
    )


def _sc_call(t3, conn, mem):
    mesh = plsc.VectorSubcoreMesh(core_axis_name="c", subcore_axis_name="s")
    return pl.kernel(
        _sc_body,
        out_type=jax.ShapeDtypeStruct((NC, 4, N_PER_SC // 4, WORDS), jnp.int32),
        name="ramlayer_sc",
        mesh=mesh,
        compiler_params=pltpu.CompilerParams(
            needs_layout_passes=False,
            use_tc_tiling_on_sc=False,
        ),
        scratch_types=[
            pltpu.VMEM_SHARED((N_PER_SC // 4, 4, NS, 2 * LANES), jnp.int32),
        ],
    )(t3, conn, mem)


def _pack4(p0, p1, p2, p3):
    return (p0.astype(jnp.int32) | (p1.astype(jnp.int32) << 8)
            | (p2.astype(jnp.int32) << 16) | (p3.astype(jnp.int32) << 24))


def kernel(input_bits, connections, memory):
    ib = input_bits.astype(jnp.uint8)
    t3w = _pack4(ib[0:256], ib[256:512], ib[512:768], ib[768:1024]).T
    mem32 = jnp.concatenate(
        [_pack4(*(memory[c * 2048 + 512 * q: c * 2048 + 512 * (q + 1)]
                  for q in range(4))) for c in range(NC)], axis=0)
    conn_p = jnp.pad(connections, ((0, 0), (0, LANES - K)))
    conn_f = conn_p.reshape(N // LANES, LANES * LANES)
    outw = _sc_call(t3w, conn_f, mem32)
    outw = outw.reshape(N, WORDS).T
    ys = [((outw >> (8 * i)) & 1).astype(jnp.bool_) for i in range(4)]
    return jnp.concatenate(ys, axis=0)


# speedup vs baseline: 1.0704x; 1.0140x over previous
import jax
import jax.numpy as jnp
from jax import lax
from jax.experimental import pallas as pl
from jax.experimental.pallas import tpu as pltpu
from jax.experimental.pallas import tpu_sc as plsc

B = 1024
J = 2048
N = 4096
K = 12
NC = 2
NS = 16
LANES = 16

N_PER_SC = N // NC
B_PER_TILE = B // NS
N_PER_TILE = N_PER_SC // NS
CONN_BLK = 256
N_CHUNK = 32
GROUPS = B // (4 * LANES)
WORDS = B // 4


def _sc_body(t3_hbm, conn_hbm, mem_hbm, out_hbm, addr_sh):
    c = lax.axis_index("c")
    s = lax.axis_index("s")
    lane = lax.iota(jnp.int32, LANES)

    def phase1(inp_v, conn_v, accbuf):
        pltpu.sync_copy(t3_hbm.at[:, pl.ds(s * LANES, LANES)], inp_v)

        def blk_body(blk, _):
            n0 = c * N_PER_SC + blk * CONN_BLK
            pltpu.sync_copy(conn_hbm.at[pl.ds(pl.multiple_of(n0 // LANES, 8), CONN_BLK // LANES), :],
                            conn_v)

            @plsc.parallel_loop(0, CONN_BLK, unroll=4)
            def n_body(nn):
                hi = jnp.zeros((LANES,), jnp.int32)
                lo = jnp.zeros((LANES,), jnp.int32)
                cvec = conn_v[nn >> 4, pl.ds((nn & 15) * LANES, LANES)]
                for k in range(K):
                    wv = inp_v[cvec[k], pl.ds(0, LANES)]
                    if k < 6:
                        hi = hi + (wv << (5 - k))
                    else:
                        lo = lo + (wv << (11 - k))
                accbuf[nn, pl.ds(0, LANES)] = hi
                accbuf[nn, pl.ds(LANES, LANES)] = lo
            pltpu.sync_copy(
                accbuf,
                addr_sh.at[pl.ds((blk & 1) * CONN_BLK, CONN_BLK), blk >> 1, s, :])
            return 0

        lax.fori_loop(0, N_PER_SC // CONN_BLK, blk_body, 0)

    pl.run_scoped(
        phase1,
        pltpu.VMEM((J, LANES), jnp.int32),
        pltpu.VMEM((CONN_BLK // LANES, LANES * LANES), jnp.int32),
        pltpu.VMEM((CONN_BLK, 2 * LANES), jnp.int32),
    )
    plsc.subcore_barrier()

    rbase = c * (N_PER_SC // 4) + s * N_CHUNK

    def phase2(mem_v, addr_v, out_v):
        def chunk_body(j, _):
            pltpu.sync_copy(
                mem_hbm.at[pl.ds(pl.multiple_of(rbase + j * 8, 8), 8), :],
                mem_v)
            pltpu.sync_copy(
                addr_sh.at[pl.ds(s * N_CHUNK + j * 8, 8), :, :, :], addr_v)

            @plsc.parallel_loop(0, N_CHUNK, unroll=2)
            def i_body(i):
                row = jnp.zeros((LANES,), jnp.int32) + (i >> 2)
                msk = 255 << ((i & 3) * 8)
                tgt = 2 << ((i & 3) * 8)

                @plsc.parallel_loop(0, GROUPS, unroll=4)
                def g_body(g):
                    hi = addr_v[i >> 2, i & 3, g, pl.ds(0, LANES)]
                    lo = addr_v[i >> 2, i & 3, g, pl.ds(LANES, LANES)]
                    out_w = jnp.zeros((LANES,), jnp.int32)
                    for bi in range(4):
                        h = (hi >> (8 * bi)) & 63
                        l = (lo >> (8 * bi)) & 63
                        a = (h << 6) | l
                        wv = plsc.load_gather(mem_v, [row, a])
                        r = ((wv & msk) == tgt).astype(jnp.int32)
                        out_w = out_w | (r << (8 * bi))
                    out_v[i & 3, i >> 2, pl.ds(g * LANES, LANES)] = out_w
            pltpu.sync_copy(
                out_v,
                out_hbm.at[c, :, pl.ds(s * N_CHUNK + j * 8, 8), :])
            return 0

        lax.fori_loop(0, N_PER_TILE // N_CHUNK, chunk_body, 0)

    pl.run_scoped(
        phase2,
        pltpu.VMEM((8, 4096), jnp.int32),
        pltpu.VMEM((8, 4, GROUPS, 2 * LANES), jnp.int32),
        pltpu.VMEM((4, 8, WORDS), jnp.int32),
    )


def _sc_call(t3, conn, mem):
    mesh = plsc.VectorSubcoreMesh(core_axis_name="c", subcore_axis_name="s")
    return pl.kernel(
        _sc_body,
        out_type=jax.ShapeDtypeStruct((NC, 4, N_PER_SC // 4, WORDS), jnp.int32),
        name="ramlayer_sc",
        mesh=mesh,
        compiler_params=pltpu.CompilerParams(
            needs_layout_passes=False,
            use_tc_tiling_on_sc=False,
        ),
        scratch_types=[
            pltpu.VMEM_SHARED((N_PER_SC // 4, 4, NS, 2 * LANES), jnp.int32),
        ],
    )(t3, conn, mem)


def _pack4(p0, p1, p2, p3):
    return (p0.astype(jnp.int32) | (p1.astype(jnp.int32) << 8)
            | (p2.astype(jnp.int32) << 16) | (p3.astype(jnp.int32) << 24))


def kernel(input_bits, connections, memory):
    ib = input_bits.astype(jnp.uint8)
    t3w = _pack4(ib[0:256], ib[256:512], ib[512:768], ib[768:1024]).T
    mem32 = jnp.concatenate(
        [_pack4(*(memory[c * 2048 + 512 * q: c * 2048 + 512 * (q + 1)]
                  for q in range(4))) for c in range(NC)], axis=0)
    conn_p = jnp.pad(connections, ((0, 0), (0, LANES - K)))
    conn_f = conn_p.reshape(N // LANES, LANES * LANES)
    outw = _sc_call(t3w, conn_f, mem32)
    outw = outw.reshape(N, WORDS).T
    ys = [((outw >> (8 * i)) & 1).astype(jnp.bool_) for i in range(4)]
    return jnp.concatenate(ys, axis=0)
